# Initial kernel scaffold; baseline (speedup 1.0000x reference)
import jax
import jax.numpy as jnp
from jax.experimental import pallas as pl


def kernel(x, edge_index, edge_weight, W1, b1, W2, b2):
    raise NotImplementedError("write your pallas kernel here")


# speedup vs baseline: 8.1061x; 8.1061x over previous
import functools

import jax
import jax.numpy as jnp
from jax import lax
from jax.experimental import pallas as pl
from jax.experimental.pallas import tpu as pltpu
from jax.experimental.pallas import tpu_sc as plsc

N = 10000
NP = 10240
E = 160000
EP = 163840
EPR = EP // 128
D_IN = 256
D_HID = 256
D_OUT = 128

_MESH = plsc.VectorSubcoreMesh(
    core_axis_name="c", subcore_axis_name="s", num_cores=2, num_subcores=16)


def _mm1_body(x_ref, w_ref, o_ref):
    o_ref[...] = lax.dot_general(
        x_ref[...], w_ref[...], (((1,), (0,)), ((), ())),
        precision=lax.Precision.HIGHEST, preferred_element_type=jnp.float32)


def _matmul1(x_p, W1):
    BN = 1280
    nb = NP // BN
    return pl.pallas_call(
        _mm1_body,
        grid=(nb, 2),
        in_specs=[
            pl.BlockSpec((BN, D_IN), lambda i, c: (i, 0)),
            pl.BlockSpec((D_IN, 128), lambda i, c: (0, c)),
        ],
        out_specs=pl.BlockSpec((BN, 128), lambda i, c: (c * nb + i, 0)),
        out_shape=jax.ShapeDtypeStruct((2 * NP, 128), jnp.float32),
    )(x_p, W1)


def _mm2_body(a_ref, b_ref, b1a_ref, b1b_ref, w2a_ref, w2b_ref, o_ref):
    ga = jnp.maximum(a_ref[...] + b1a_ref[0, 0], 0.0)
    gb = jnp.maximum(b_ref[...] + b1b_ref[0, 0], 0.0)
    oa = lax.dot_general(ga, w2a_ref[0], (((1,), (0,)), ((), ())),
                         precision=lax.Precision.HIGHEST,
                         preferred_element_type=jnp.float32)
    ob = lax.dot_general(gb, w2b_ref[0], (((1,), (0,)), ((), ())),
                         precision=lax.Precision.HIGHEST,
                         preferred_element_type=jnp.float32)
    o_ref[...] = oa + ob


def _matmul2(a1cat, b1r, W2r):
    BN = 1280
    nb = NP // BN
    return pl.pallas_call(
        _mm2_body,
        grid=(nb,),
        in_specs=[
            pl.BlockSpec((BN, 128), lambda i: (i, 0)),
            pl.BlockSpec((BN, 128), lambda i: (nb + i, 0)),
            pl.BlockSpec((1, 1, 128), lambda i: (0, 0, 0)),
            pl.BlockSpec((1, 1, 128), lambda i: (1, 0, 0)),
            pl.BlockSpec((1, 128, 128), lambda i: (0, 0, 0)),
            pl.BlockSpec((1, 128, 128), lambda i: (1, 0, 0)),
        ],
        out_specs=pl.BlockSpec((BN, 128), lambda i: (i, 0)),
        out_shape=jax.ShapeDtypeStruct((NP, 128), jnp.float32),
    )(a1cat, a1cat, b1r, b1r, W2r, W2r)


def _sum_body(p0_ref, p1_ref, b2_ref, o_ref):
    o_ref[...] = p0_ref[...] + p1_ref[...] + b2_ref[0, 0]


def _sum_tc(parts, b2r):
    BN = 1280
    nb = NP // BN
    return pl.pallas_call(
        _sum_body,
        grid=(nb,),
        in_specs=[
            pl.BlockSpec((BN, 128), lambda i: (i, 0)),
            pl.BlockSpec((BN, 128), lambda i: (nb + i, 0)),
            pl.BlockSpec((1, 1, 128), lambda i: (0, 0, 0)),
        ],
        out_specs=pl.BlockSpec((BN, 128), lambda i: (i, 0)),
        out_shape=jax.ShapeDtypeStruct((NP, 128), jnp.float32),
    )(parts, parts, b2r)


@functools.partial(
    pl.kernel,
    out_type=jax.ShapeDtypeStruct((2, NP), jnp.float32),
    mesh=_MESH,
    compiler_params=pltpu.CompilerParams(needs_layout_passes=False),
    scratch_types=[
        pltpu.VMEM_SHARED((NP,), jnp.float32),
        pltpu.VMEM((40, 128), jnp.int32),
        pltpu.VMEM((EP // 32,), jnp.float32),
        pltpu.VMEM((640,), jnp.float32),
    ],
)
def _deg_partial_kernel(dstm_h, w1_h, deg_h, deg_s, dstv, wv, degv):
    c = lax.axis_index("c")
    s = lax.axis_index("s")

    z16 = jnp.zeros((16,), jnp.float32)

    def _zero(i, carry):
        degv[pl.ds(i * 16, 16)] = z16
        return carry
    lax.fori_loop(0, 40, _zero, 0)
    pltpu.sync_copy(degv, deg_s.at[pl.ds(s * 640, 640)])
    plsc.subcore_barrier()

    wid = c * 16 + s
    pltpu.sync_copy(dstm_h.at[pl.ds(wid * 40, 40)], dstv)
    pltpu.sync_copy(w1_h.at[pl.ds(wid * (EP // 32), EP // 32)], wv)

    def _acc(j, carry):
        pltpu.sync_copy(wv.at[pl.ds(j * 128, 128)],
                        deg_s.at[dstv.at[j]], add=True)
        return carry
    lax.fori_loop(0, 40, _acc, 0)
    plsc.subcore_barrier()

    pltpu.sync_copy(deg_s.at[pl.ds(s * 640, 640)], degv)
    pltpu.sync_copy(degv, deg_h.at[c, pl.ds(s * 640, 640)])


def _dis_body(deg_ref, dis_ref):
    deg = deg_ref[0] + deg_ref[1]
    dis_ref[...] = jnp.where(deg > 0.0, lax.rsqrt(deg), 0.0)


def _dis_tc(deg_parts):
    return pl.pallas_call(
        _dis_body,
        out_shape=jax.ShapeDtypeStruct((80, 128), jnp.float32),
    )(deg_parts.reshape(2, 80, 128))


@functools.partial(
    pl.kernel,
    out_type=jax.ShapeDtypeStruct((EP,), jnp.float32),
    mesh=_MESH,
    compiler_params=pltpu.CompilerParams(needs_layout_passes=False),
    scratch_types=[
        pltpu.VMEM((NP,), jnp.float32),
        pltpu.VMEM((EP // 32,), jnp.int32),
        pltpu.VMEM((EP // 32,), jnp.int32),
        pltpu.VMEM((EP // 32,), jnp.float32),
        pltpu.VMEM((EP // 32,), jnp.float32),
    ],
)
def _norm_kernel(src1_h, dst1_h, w1_h, dis_h, norm_h,
                 disv, srcv2, dstv2, wv2, normv):
    c = lax.axis_index("c")
    s = lax.axis_index("s")
    epr32 = EP // 32

    pltpu.sync_copy(dis_h, disv)
    wid = c * 16 + s
    base = wid * epr32
    pltpu.sync_copy(src1_h.at[pl.ds(base, epr32)], srcv2)
    pltpu.sync_copy(dst1_h.at[pl.ds(base, epr32)], dstv2)
    pltpu.sync_copy(w1_h.at[pl.ds(base, epr32)], wv2)

    def _nrm(r, carry):
        for g in range(8):
            off = r * 128 + g * 16
            s16 = srcv2[pl.ds(off, 16)]
            d16 = dstv2[pl.ds(off, 16)]
            gs = plsc.load_gather(disv, [s16])
            gd = plsc.load_gather(disv, [d16])
            normv[pl.ds(off, 16)] = gs * wv2[pl.ds(off, 16)] * gd
        return carry
    lax.fori_loop(0, epr32 // 128, _nrm, 0)
    pltpu.sync_copy(normv, norm_h.at[pl.ds(base, epr32)])


def _make_agg(col_split):
    eh = EP // 32
    n_phases = 2 if col_split else 1

    scratch = [
        pltpu.VMEM_SHARED((NP, 128), jnp.float32),
        pltpu.VMEM((eh,), jnp.int32),
        pltpu.VMEM((eh,), jnp.float32),
        pltpu.VMEM((40, 128), jnp.int32),
        pltpu.VMEM((128, 128), jnp.float32),
        pltpu.VMEM((16, 128), jnp.float32),
        pltpu.SemaphoreType.DMA,
    ]

    def body(h_h, src1_h, dstm_h, norm1_h, out_h,
             acc, srcv, normv, dstv, rows, initbuf, sem):
        c = lax.axis_index("c")
        s = lax.axis_index("s")
        coff = c * NP if col_split else c * 0

        z16 = jnp.zeros((16,), jnp.float32)
        for i in range(16):
            for g in range(8):
                initbuf[i, pl.ds(g * 16, 16)] = z16
        for k in range(40):
            pltpu.sync_copy(initbuf, acc.at[pl.ds(s * 640 + k * 16, 16)])
        plsc.subcore_barrier()

        for p in range(n_phases):
            if col_split:
                be = s * (EP // 16) + p * eh
                br = s * 80 + p * 40
            else:
                be = c * (EP // 2) + s * eh
                br = c * 640 + s * 40
            pltpu.sync_copy(src1_h.at[pl.ds(be, eh)], srcv)
            pltpu.sync_copy(norm1_h.at[pl.ds(be, eh)], normv)
            pltpu.sync_copy(dstm_h.at[pl.ds(br, 40)], dstv)

            if col_split:
                def _off(r, carry):
                    for g in range(8):
                        o = r * 128 + g * 16
                        srcv[pl.ds(o, 16)] = srcv[pl.ds(o, 16)] + coff
                    return carry
                lax.fori_loop(0, 40, _off, 0)

            def _chunk(j, carry):
                pltpu.async_copy(
                    h_h.at[srcv.at[pl.ds(j * 128, 128)]], rows, sem).wait()

                def _scale(r, carry2):
                    nsp = plsc.load_gather(
                        normv, [jnp.full((16,), j * 128 + r, jnp.int32)])
                    for g in range(8):
                        rows[r, pl.ds(g * 16, 16)] = (
                            rows[r, pl.ds(g * 16, 16)] * nsp)
                    return carry2
                lax.fori_loop(0, 128, _scale, 0)

                pltpu.sync_copy(rows, acc.at[dstv.at[j]], add=True)
                return carry
            lax.fori_loop(0, 40, _chunk, 0)
        plsc.subcore_barrier()

        for k in range(5):
            r0 = s * 640 + k * 128
            pltpu.sync_copy(acc.at[pl.ds(r0, 128)],
                            out_h.at[pl.ds(c * NP + r0, 128)])

    return pl.kernel(
        body,
        out_type=jax.ShapeDtypeStruct((2 * NP, 128), jnp.float32),
        mesh=_MESH,
        scratch_types=scratch,
        compiler_params=pltpu.CompilerParams(needs_layout_passes=False),
    )


_agg1 = _make_agg(col_split=True)
_agg2 = _make_agg(col_split=False)


def kernel(x, edge_index, edge_weight, W1, b1, W2, b2):
    src = edge_index[0].astype(jnp.int32)
    dst = edge_index[1].astype(jnp.int32)
    npad = EP - E
    pad_idx = (jnp.arange(npad, dtype=jnp.int32) * 37) % N
    src1 = jnp.concatenate([src, pad_idx])
    dst1 = jnp.concatenate([dst, pad_idx])
    w1 = jnp.concatenate([edge_weight, jnp.zeros((npad,), jnp.float32)])
    dstm = dst1.reshape(EPR, 128)
    x_p = jnp.pad(x, ((0, NP - N), (0, 0)))
    b1r = b1.reshape(2, 1, 128)
    W2r = W2.reshape(2, 128, 128)
    b2r = b2.reshape(1, 1, 128)

    h1cat = _matmul1(x_p, W1)
    deg_parts = _deg_partial_kernel(dstm, w1)
    dis = _dis_tc(deg_parts).reshape(NP)
    norm1 = _norm_kernel(src1, dst1, w1, dis)
    a1cat = _agg1(h1cat, src1, dstm, norm1)
    h2 = _matmul2(a1cat, b1r, W2r)
    parts2 = _agg2(h2, src1, dstm, norm1)
    z = _sum_tc(parts2, b2r)
    return z[:N]


# speedup vs baseline: 11.4308x; 1.4101x over previous
import functools

import jax
import jax.numpy as jnp
from jax import lax
from jax.experimental import pallas as pl
from jax.experimental.pallas import tpu as pltpu
from jax.experimental.pallas import tpu_sc as plsc

N = 10000
NP = 10240
E = 160000
EP = 163840
EPR = EP // 128
D_IN = 256
D_HID = 256
D_OUT = 128

_MESH = plsc.VectorSubcoreMesh(
    core_axis_name="c", subcore_axis_name="s", num_cores=2, num_subcores=16)


def _mm1_body(x_ref, w_ref, o_ref):
    o_ref[...] = lax.dot_general(
        x_ref[...], w_ref[...], (((1,), (0,)), ((), ())),
        precision=lax.Precision.HIGHEST, preferred_element_type=jnp.float32)


def _matmul1(x_p, W1):
    BN = 1280
    nb = NP // BN
    return pl.pallas_call(
        _mm1_body,
        grid=(nb, 2),
        in_specs=[
            pl.BlockSpec((BN, D_IN), lambda i, c: (i, 0)),
            pl.BlockSpec((D_IN, 128), lambda i, c: (0, c)),
        ],
        out_specs=pl.BlockSpec((BN, 128), lambda i, c: (c * nb + i, 0)),
        out_shape=jax.ShapeDtypeStruct((2 * NP, 128), jnp.float32),
    )(x_p, W1)


def _mm2_body(a_ref, b_ref, b1a_ref, b1b_ref, w2a_ref, w2b_ref, o_ref):
    ga = jnp.maximum(a_ref[...] + b1a_ref[0, 0], 0.0)
    gb = jnp.maximum(b_ref[...] + b1b_ref[0, 0], 0.0)
    oa = lax.dot_general(ga, w2a_ref[0], (((1,), (0,)), ((), ())),
                         precision=lax.Precision.HIGHEST,
                         preferred_element_type=jnp.float32)
    ob = lax.dot_general(gb, w2b_ref[0], (((1,), (0,)), ((), ())),
                         precision=lax.Precision.HIGHEST,
                         preferred_element_type=jnp.float32)
    o_ref[...] = oa + ob


def _matmul2(a1cat, b1r, W2r):
    BN = 1280
    nb = NP // BN
    return pl.pallas_call(
        _mm2_body,
        grid=(nb,),
        in_specs=[
            pl.BlockSpec((BN, 128), lambda i: (i, 0)),
            pl.BlockSpec((BN, 128), lambda i: (nb + i, 0)),
            pl.BlockSpec((1, 1, 128), lambda i: (0, 0, 0)),
            pl.BlockSpec((1, 1, 128), lambda i: (1, 0, 0)),
            pl.BlockSpec((1, 128, 128), lambda i: (0, 0, 0)),
            pl.BlockSpec((1, 128, 128), lambda i: (1, 0, 0)),
        ],
        out_specs=pl.BlockSpec((BN, 128), lambda i: (i, 0)),
        out_shape=jax.ShapeDtypeStruct((NP, 128), jnp.float32),
    )(a1cat, a1cat, b1r, b1r, W2r, W2r)


def _sum_body(p0_ref, p1_ref, b2_ref, o_ref):
    o_ref[...] = p0_ref[...] + p1_ref[...] + b2_ref[0, 0]


def _sum_tc(parts, b2r):
    BN = 1280
    nb = NP // BN
    return pl.pallas_call(
        _sum_body,
        grid=(nb,),
        in_specs=[
            pl.BlockSpec((BN, 128), lambda i: (i, 0)),
            pl.BlockSpec((BN, 128), lambda i: (nb + i, 0)),
            pl.BlockSpec((1, 1, 128), lambda i: (0, 0, 0)),
        ],
        out_specs=pl.BlockSpec((BN, 128), lambda i: (i, 0)),
        out_shape=jax.ShapeDtypeStruct((NP, 128), jnp.float32),
    )(parts, parts, b2r)


@functools.partial(
    pl.kernel,
    out_type=jax.ShapeDtypeStruct((2, NP), jnp.float32),
    mesh=_MESH,
    compiler_params=pltpu.CompilerParams(needs_layout_passes=False),
    scratch_types=[
        pltpu.VMEM_SHARED((NP,), jnp.float32),
        pltpu.VMEM((40, 128), jnp.int32),
        pltpu.VMEM((EP // 32,), jnp.float32),
        pltpu.VMEM((640,), jnp.float32),
    ],
)
def _deg_partial_kernel(dstm_h, w1_h, deg_h, deg_s, dstv, wv, degv):
    c = lax.axis_index("c")
    s = lax.axis_index("s")

    z16 = jnp.zeros((16,), jnp.float32)

    def _zero(i, carry):
        degv[pl.ds(i * 16, 16)] = z16
        return carry
    lax.fori_loop(0, 40, _zero, 0)
    pltpu.sync_copy(degv, deg_s.at[pl.ds(s * 640, 640)])
    plsc.subcore_barrier()

    wid = c * 16 + s
    pltpu.sync_copy(dstm_h.at[pl.ds(wid * 40, 40)], dstv)
    pltpu.sync_copy(w1_h.at[pl.ds(wid * (EP // 32), EP // 32)], wv)

    def _acc(j, carry):
        pltpu.sync_copy(wv.at[pl.ds(j * 128, 128)],
                        deg_s.at[dstv.at[j]], add=True)
        return carry
    lax.fori_loop(0, 40, _acc, 0)
    plsc.subcore_barrier()

    pltpu.sync_copy(deg_s.at[pl.ds(s * 640, 640)], degv)
    pltpu.sync_copy(degv, deg_h.at[c, pl.ds(s * 640, 640)])


def _dis_body(deg_ref, dis_ref):
    deg = deg_ref[0] + deg_ref[1]
    dis_ref[...] = jnp.where(deg > 0.0, lax.rsqrt(deg), 0.0)


def _dis_tc(deg_parts):
    return pl.pallas_call(
        _dis_body,
        out_shape=jax.ShapeDtypeStruct((80, 128), jnp.float32),
    )(deg_parts.reshape(2, 80, 128))


@functools.partial(
    pl.kernel,
    out_type=jax.ShapeDtypeStruct((EP,), jnp.float32),
    mesh=_MESH,
    compiler_params=pltpu.CompilerParams(needs_layout_passes=False),
    scratch_types=[
        pltpu.VMEM((NP,), jnp.float32),
        pltpu.VMEM((EP // 32,), jnp.int32),
        pltpu.VMEM((EP // 32,), jnp.int32),
        pltpu.VMEM((EP // 32,), jnp.float32),
        pltpu.VMEM((EP // 32,), jnp.float32),
    ],
)
def _norm_kernel(src1_h, dst1_h, w1_h, dis_h, norm_h,
                 disv, srcv2, dstv2, wv2, normv):
    c = lax.axis_index("c")
    s = lax.axis_index("s")
    epr32 = EP // 32

    pltpu.sync_copy(dis_h, disv)
    wid = c * 16 + s
    base = wid * epr32
    pltpu.sync_copy(src1_h.at[pl.ds(base, epr32)], srcv2)
    pltpu.sync_copy(dst1_h.at[pl.ds(base, epr32)], dstv2)
    pltpu.sync_copy(w1_h.at[pl.ds(base, epr32)], wv2)

    def _nrm(r, carry):
        for g in range(8):
            off = r * 128 + g * 16
            s16 = srcv2[pl.ds(off, 16)]
            d16 = dstv2[pl.ds(off, 16)]
            gs = plsc.load_gather(disv, [s16])
            gd = plsc.load_gather(disv, [d16])
            normv[pl.ds(off, 16)] = gs * wv2[pl.ds(off, 16)] * gd
        return carry
    lax.fori_loop(0, epr32 // 128, _nrm, 0)
    pltpu.sync_copy(normv, norm_h.at[pl.ds(base, epr32)])


def _make_agg(col_split):
    eh = EP // 32
    n_phases = 2 if col_split else 1

    scratch = [
        pltpu.VMEM_SHARED((NP, 128), jnp.float32),
        pltpu.VMEM((eh,), jnp.int32),
        pltpu.VMEM((eh,), jnp.float32),
        pltpu.VMEM((40, 128), jnp.int32),
        pltpu.VMEM((128, 128), jnp.float32),
        pltpu.VMEM((128, 128), jnp.float32),
        pltpu.SemaphoreType.DMA,
        pltpu.SemaphoreType.DMA,
    ]

    def body(h_h, src1_h, dstm_h, norm1_h, out_h,
             acc, srcv, normv, dstv, rows0, rows1, gsem0, gsem1):
        c = lax.axis_index("c")
        s = lax.axis_index("s")
        coff = c * NP if col_split else c * 0

        z16 = jnp.zeros((16,), jnp.float32)
        for i in range(16):
            for g in range(8):
                rows0[i, pl.ds(g * 16, 16)] = z16
        for k in range(40):
            pltpu.sync_copy(rows0.at[pl.ds(0, 16)],
                            acc.at[pl.ds(s * 640 + k * 16, 16)])
        plsc.subcore_barrier()

        def _gather(j, rows, sem):
            return pltpu.make_async_copy(
                h_h.at[srcv.at[pl.ds(j * 128, 128)]], rows, sem)

        def _consume(j, rows):
            def _scale(r, carry2):
                nsp = plsc.load_gather(
                    normv, [jnp.full((16,), j * 128 + r, jnp.int32)])
                for g in range(8):
                    rows[r, pl.ds(g * 16, 16)] = (
                        rows[r, pl.ds(g * 16, 16)] * nsp)
                return carry2
            lax.fori_loop(0, 128, _scale, 0)
            pltpu.sync_copy(rows, acc.at[dstv.at[j]], add=True)

        for p in range(2 if col_split else 1):
            if col_split:
                be = s * (EP // 16) + p * eh
                br = s * 80 + p * 40
            else:
                be = c * (EP // 2) + s * eh
                br = c * 640 + s * 40
            pltpu.sync_copy(src1_h.at[pl.ds(be, eh)], srcv)
            pltpu.sync_copy(norm1_h.at[pl.ds(be, eh)], normv)
            pltpu.sync_copy(dstm_h.at[pl.ds(br, 40)], dstv)

            if col_split:
                def _off(r, carry):
                    for g in range(8):
                        o = r * 128 + g * 16
                        srcv[pl.ds(o, 16)] = srcv[pl.ds(o, 16)] + coff
                    return carry
                lax.fori_loop(0, 40, _off, 0)

            _gather(0, rows0, gsem0).start()

            def _pair(i, carry):
                a = 2 * i
                _gather(a + 1, rows1, gsem1).start()
                _gather(a, rows0, gsem0).wait()
                _consume(a, rows0)

                @pl.when(i < 19)
                def _():
                    _gather(a + 2, rows0, gsem0).start()
                _gather(a + 1, rows1, gsem1).wait()
                _consume(a + 1, rows1)
                return carry
            lax.fori_loop(0, 20, _pair, 0)
        plsc.subcore_barrier()

        for k in range(5):
            r0 = s * 640 + k * 128
            pltpu.sync_copy(acc.at[pl.ds(r0, 128)],
                            out_h.at[pl.ds(c * NP + r0, 128)])

    return pl.kernel(
        body,
        out_type=jax.ShapeDtypeStruct((2 * NP, 128), jnp.float32),
        mesh=_MESH,
        scratch_types=scratch,
        compiler_params=pltpu.CompilerParams(needs_layout_passes=False),
    )


_agg1 = _make_agg(col_split=True)
_agg2 = _make_agg(col_split=False)


def kernel(x, edge_index, edge_weight, W1, b1, W2, b2):
    src = edge_index[0].astype(jnp.int32)
    dst = edge_index[1].astype(jnp.int32)
    npad = EP - E
    pad_idx = (jnp.arange(npad, dtype=jnp.int32) * 37) % N
    src1 = jnp.concatenate([src, pad_idx])
    dst1 = jnp.concatenate([dst, pad_idx])
    w1 = jnp.concatenate([edge_weight, jnp.zeros((npad,), jnp.float32)])
    dstm = dst1.reshape(EPR, 128)
    x_p = jnp.pad(x, ((0, NP - N), (0, 0)))
    b1r = b1.reshape(2, 1, 128)
    W2r = W2.reshape(2, 128, 128)
    b2r = b2.reshape(1, 1, 128)

    h1cat = _matmul1(x_p, W1)
    deg_parts = _deg_partial_kernel(dstm, w1)
    dis = _dis_tc(deg_parts).reshape(NP)
    norm1 = _norm_kernel(src1, dst1, w1, dis)
    a1cat = _agg1(h1cat, src1, dstm, norm1)
    h2 = _matmul2(a1cat, b1r, W2r)
    parts2 = _agg2(h2, src1, dstm, norm1)
    z = _sum_tc(parts2, b2r)
    return z[:N]
